# baseline (speedup 1.0000x reference)
import dataclasses
import functools

import jax
import jax.numpy as jnp
from jax import lax
from jax.experimental import pallas as pl
from jax.experimental.pallas import tpu as pltpu
from jax.experimental.pallas import tpu_sc as plsc

N_NODES = 10000
N_EDGES = 320000
D = 128
CLS = 10
N_GRAPHS = 64

NC = 2
NS = 16
EDGES_PER_CORE = N_EDGES // NC
EDGES_PER_TILE = EDGES_PER_CORE // NS
BLK = 80
NBLK = EDGES_PER_TILE // BLK
N_PAD = 10240
ROWS_PER_TILE = N_PAD // NS

_sc_mesh = plsc.VectorSubcoreMesh(core_axis_name="c", subcore_axis_name="s")

_sc_cp = pltpu.CompilerParams()
if "needs_layout_passes" in pltpu.CompilerParams.__dataclass_fields__:
    _sc_cp = dataclasses.replace(_sc_cp, needs_layout_passes=False)


L = 16
NGROUP = EDGES_PER_TILE // L


@functools.partial(
    pl.kernel,
    out_type=jax.ShapeDtypeStruct((NC * N_PAD,), jnp.float32),
    mesh=_sc_mesh,
    compiler_params=_sc_cp,
    scratch_types=[
        pltpu.VMEM((EDGES_PER_TILE,), jnp.int32),
        pltpu.VMEM((N_PAD,), jnp.float32),
        pltpu.VMEM((L, ROWS_PER_TILE), jnp.float32),
        pltpu.VMEM((ROWS_PER_TILE,), jnp.float32),
        pltpu.VMEM_SHARED((NS, N_PAD), jnp.float32),
    ],
)
def _degree_sc(dst_hbm, out_hbm, dst_t, acc_t, red_in, red_out, shr):
    c = lax.axis_index("c")
    s = lax.axis_index("s")
    base = c * EDGES_PER_CORE + s * EDGES_PER_TILE
    pltpu.sync_copy(dst_hbm.at[pl.ds(base, EDGES_PER_TILE)], dst_t)

    @pl.loop(0, N_PAD, step=L)
    def _(i):
        acc_t[pl.ds(i, L)] = jnp.zeros((L,), jnp.float32)

    ones_v = jnp.full((L,), 1.0, jnp.float32)

    @pl.loop(0, NGROUP)
    def _(g):
        idx = dst_t[pl.ds(g * L, L)]
        plsc.addupdate_scatter(acc_t, [idx], ones_v)

    pltpu.sync_copy(acc_t, shr.at[s])
    plsc.subcore_barrier()

    col0 = s * ROWS_PER_TILE
    pltpu.sync_copy(shr.at[:, pl.ds(col0, ROWS_PER_TILE)], red_in)

    @pl.loop(0, ROWS_PER_TILE, step=L)
    def _(w):
        tot = jnp.zeros((L,), jnp.float32)
        for r in range(NS):
            tot = tot + red_in[r, pl.ds(w, L)]
        red_out[pl.ds(w, L)] = tot

    pltpu.sync_copy(red_out, out_hbm.at[pl.ds(c * N_PAD + col0, ROWS_PER_TILE)])


GBLK = 128
NFULL = EDGES_PER_TILE // GBLK
TAIL = EDGES_PER_TILE - NFULL * GBLK
NSLOT = 2
NITER = NFULL // NSLOT - 1


@functools.partial(
    pl.kernel,
    out_type=jax.ShapeDtypeStruct((NC * N_PAD, D), jnp.float32),
    mesh=_sc_mesh,
    scratch_types=(
        [pltpu.VMEM((1, GBLK), jnp.int32)] * (2 * NSLOT)
        + [pltpu.VMEM((1, TAIL), jnp.int32)] * 2
        + [pltpu.VMEM((GBLK, D), jnp.float32)] * NSLOT
        + [pltpu.VMEM_SHARED((N_PAD, D), jnp.float32)]
        + [pltpu.SemaphoreType.DMA] * NSLOT
    ),
)
def _agg_sc(hs_hbm, src_hbm, dst_hbm, zeros_hbm, out_hbm,
            si0, di0, si1, di1, sit, dit,
            rows0, rows1, acc_sh, g0, g1):
    c = lax.axis_index("c")
    s = lax.axis_index("s")
    r0 = s * ROWS_PER_TILE
    pltpu.sync_copy(zeros_hbm.at[pl.ds(r0, ROWS_PER_TILE)],
                    acc_sh.at[pl.ds(r0, ROWS_PER_TILE)])
    plsc.subcore_barrier()
    base = c * EDGES_PER_CORE + s * EDGES_PER_TILE
    slots = ((si0, di0, rows0, g0), (si1, di1, rows1, g1))

    def fetch_idx(si, di, j):
        e0 = base + j * GBLK
        pltpu.sync_copy(src_hbm.at[pl.ds(e0, GBLK)], si.at[0])
        pltpu.sync_copy(dst_hbm.at[pl.ds(e0, GBLK)], di.at[0])

    def start_gather(si, rows, sem):
        pltpu.async_copy(hs_hbm.at[si.at[0]], rows, sem)

    def wait_gather(si, rows, sem):
        pltpu.make_async_copy(hs_hbm.at[si.at[0]], rows, sem).wait()

    for b, (si, di, rows, sem) in enumerate(slots):
        fetch_idx(si, di, b)
        start_gather(si, rows, sem)

    @pl.loop(0, NITER)
    def _(k):
        j = NSLOT * k
        for b, (si, di, rows, sem) in enumerate(slots):
            wait_gather(si, rows, sem)
            pltpu.sync_copy(rows, acc_sh.at[di.at[0]], add=True)
            fetch_idx(si, di, j + NSLOT + b)
            start_gather(si, rows, sem)

    for si, di, rows, sem in slots:
        wait_gather(si, rows, sem)
        pltpu.sync_copy(rows, acc_sh.at[di.at[0]], add=True)

    e0 = base + NFULL * GBLK
    pltpu.sync_copy(src_hbm.at[pl.ds(e0, TAIL)], sit.at[0])
    pltpu.sync_copy(dst_hbm.at[pl.ds(e0, TAIL)], dit.at[0])
    pltpu.async_copy(hs_hbm.at[sit.at[0]], rows0.at[pl.ds(0, TAIL)], g0).wait()
    pltpu.sync_copy(rows0.at[pl.ds(0, TAIL)], acc_sh.at[dit.at[0]], add=True)

    plsc.subcore_barrier()
    pltpu.sync_copy(acc_sh.at[pl.ds(r0, ROWS_PER_TILE)],
                    out_hbm.at[pl.ds(c * N_PAD + r0, ROWS_PER_TILE)])


R = 1280
GRID = -(-N_NODES // R)
OFF = N_PAD // R


def _mm1_body(x_ref, w_ref, g_ref):
    g_ref[...] = jnp.dot(x_ref[...], w_ref[...],
                         preferred_element_type=jnp.float32)


_mm1_call = pl.pallas_call(
    _mm1_body,
    grid=(GRID,),
    in_specs=[
        pl.BlockSpec((R, D), lambda i: (i, 0)),
        pl.BlockSpec((D, D), lambda i: (0, 0)),
    ],
    out_specs=pl.BlockSpec((R, D), lambda i: (i, 0)),
    out_shape=jax.ShapeDtypeStruct((N_NODES, D), jnp.float32),
)


def _hs1_body(g_ref, p0_ref, p1_ref, hs_ref, dinv_ref):
    deg = p0_ref[...] + p1_ref[...] + 1.0
    dinv = lax.rsqrt(deg)
    dinv_ref[...] = dinv
    hs_ref[...] = g_ref[...] * dinv


_hs1_call = pl.pallas_call(
    _hs1_body,
    grid=(GRID,),
    in_specs=[
        pl.BlockSpec((R, D), lambda i: (i, 0)),
        pl.BlockSpec((R, 1), lambda i: (i, 0)),
        pl.BlockSpec((R, 1), lambda i: (OFF + i, 0)),
    ],
    out_specs=[
        pl.BlockSpec((R, D), lambda i: (i, 0)),
        pl.BlockSpec((R, 1), lambda i: (i, 0)),
    ],
    out_shape=[
        jax.ShapeDtypeStruct((N_NODES, D), jnp.float32),
        jax.ShapeDtypeStruct((N_NODES, 1), jnp.float32),
    ],
)


def _mid_body(a0_ref, a1_ref, hs_ref, dinv_ref, w_ref, b_ref, out_ref):
    dinv = dinv_ref[...]
    pre = dinv * (a0_ref[...] + a1_ref[...] + hs_ref[...]) + b_ref[...]
    f = jnp.maximum(pre, 0.0)
    out_ref[...] = jnp.dot(f, w_ref[...],
                           preferred_element_type=jnp.float32) * dinv


_mid_call = pl.pallas_call(
    _mid_body,
    grid=(GRID,),
    in_specs=[
        pl.BlockSpec((R, D), lambda i: (i, 0)),
        pl.BlockSpec((R, D), lambda i: (OFF + i, 0)),
        pl.BlockSpec((R, D), lambda i: (i, 0)),
        pl.BlockSpec((R, 1), lambda i: (i, 0)),
        pl.BlockSpec((D, D), lambda i: (0, 0)),
        pl.BlockSpec((1, D), lambda i: (0, 0)),
    ],
    out_specs=pl.BlockSpec((R, D), lambda i: (i, 0)),
    out_shape=jax.ShapeDtypeStruct((N_NODES, D), jnp.float32),
)


def _final_body(a0_ref, a1_ref, hs_ref, dinv_ref, b_ref, batch_ref,
                wfc_ref, bfc_ref, out_ref, pooled_ref, sums, cnts):
    i = pl.program_id(0)

    @pl.when(i == 0)
    def _init():
        sums[...] = jnp.zeros_like(sums)
        cnts[...] = jnp.zeros_like(cnts)

    dinv = dinv_ref[...]
    pre = dinv * (a0_ref[...] + a1_ref[...] + hs_ref[...]) + b_ref[...]
    h2 = jnp.maximum(pre, 0.0)
    rid = i * R + lax.broadcasted_iota(jnp.int32, (R, 1), 0)
    valid = rid < N_NODES
    h2 = jnp.where(valid, h2, 0.0)
    onehot = ((batch_ref[...] ==
               lax.broadcasted_iota(jnp.int32, (R, N_GRAPHS), 1)) & valid
              ).astype(jnp.float32)
    dn = (((0,), (0,)), ((), ()))
    sums[...] += lax.dot_general(onehot, h2, dn,
                                 preferred_element_type=jnp.float32)
    cnts[...] += lax.dot_general(onehot, jnp.ones((R, D), jnp.float32), dn,
                                 preferred_element_type=jnp.float32)

    @pl.when(i == pl.num_programs(0) - 1)
    def _fini():
        pooled = sums[...] / jnp.maximum(cnts[...], 1.0)
        pooled_ref[...] = pooled
        out_ref[...] = jnp.dot(pooled, wfc_ref[...],
                               preferred_element_type=jnp.float32) + bfc_ref[...]


_final_call = pl.pallas_call(
    _final_body,
    grid=(GRID,),
    in_specs=[
        pl.BlockSpec((R, D), lambda i: (i, 0)),
        pl.BlockSpec((R, D), lambda i: (OFF + i, 0)),
        pl.BlockSpec((R, D), lambda i: (i, 0)),
        pl.BlockSpec((R, 1), lambda i: (i, 0)),
        pl.BlockSpec((1, D), lambda i: (0, 0)),
        pl.BlockSpec((R, 1), lambda i: (i, 0)),
        pl.BlockSpec((D, CLS), lambda i: (0, 0)),
        pl.BlockSpec((1, CLS), lambda i: (0, 0)),
    ],
    out_specs=[
        pl.BlockSpec((N_GRAPHS, CLS), lambda i: (0, 0)),
        pl.BlockSpec((N_GRAPHS, D), lambda i: (0, 0)),
    ],
    out_shape=[
        jax.ShapeDtypeStruct((N_GRAPHS, CLS), jnp.float32),
        jax.ShapeDtypeStruct((N_GRAPHS, D), jnp.float32),
    ],
    scratch_shapes=[
        pltpu.VMEM((N_GRAPHS, D), jnp.float32),
        pltpu.VMEM((N_GRAPHS, D), jnp.float32),
    ],
)


def kernel(x, edge_index, batch, W1, b1, W2, b2, Wfc, bfc):
    src = edge_index[0].astype(jnp.int32)
    dst = edge_index[1].astype(jnp.int32)
    batch2 = batch.astype(jnp.int32).reshape(N_NODES, 1)
    zeros_nd = jnp.zeros((N_PAD, D), jnp.float32)

    g1 = _mm1_call(x, W1)
    degp = _degree_sc(dst).reshape(NC * N_PAD, 1)
    hs1, dinv = _hs1_call(g1, degp, degp)
    agg1 = _agg_sc(hs1, src, dst, zeros_nd)
    hs2 = _mid_call(agg1, agg1, hs1, dinv, W2, b1.reshape(1, D))
    agg2 = _agg_sc(hs2, src, dst, zeros_nd)
    output, pooled = _final_call(agg2, agg2, hs2, dinv,
                                 b2.reshape(1, D), batch2,
                                 Wfc, bfc.reshape(1, CLS))
    return (output, pooled)


# speedup vs baseline: 1.0000x; 1.0000x over previous
import dataclasses
import functools

import jax
import jax.numpy as jnp
from jax import lax
from jax.experimental import pallas as pl
from jax.experimental.pallas import tpu as pltpu
from jax.experimental.pallas import tpu_sc as plsc

N_NODES = 10000
N_EDGES = 320000
D = 128
CLS = 10
N_GRAPHS = 64

NC = 2
NS = 16
EDGES_PER_CORE = N_EDGES // NC
EDGES_PER_TILE = EDGES_PER_CORE // NS
BLK = 80
NBLK = EDGES_PER_TILE // BLK
N_PAD = 10240
ROWS_PER_TILE = N_PAD // NS

_sc_mesh = plsc.VectorSubcoreMesh(core_axis_name="c", subcore_axis_name="s")

_sc_cp = pltpu.CompilerParams()
if "needs_layout_passes" in pltpu.CompilerParams.__dataclass_fields__:
    _sc_cp = dataclasses.replace(_sc_cp, needs_layout_passes=False)


L = 16
NGROUP = EDGES_PER_TILE // L


@functools.partial(
    pl.kernel,
    out_type=jax.ShapeDtypeStruct((NC * N_PAD,), jnp.float32),
    mesh=_sc_mesh,
    compiler_params=_sc_cp,
    scratch_types=[
        pltpu.VMEM((EDGES_PER_TILE,), jnp.int32),
        pltpu.VMEM((N_PAD,), jnp.float32),
        pltpu.VMEM((L, ROWS_PER_TILE), jnp.float32),
        pltpu.VMEM((ROWS_PER_TILE,), jnp.float32),
        pltpu.VMEM_SHARED((NS, N_PAD), jnp.float32),
    ],
)
def _degree_sc(dst_hbm, out_hbm, dst_t, acc_t, red_in, red_out, shr):
    c = lax.axis_index("c")
    s = lax.axis_index("s")
    base = c * EDGES_PER_CORE + s * EDGES_PER_TILE
    pltpu.sync_copy(dst_hbm.at[pl.ds(base, EDGES_PER_TILE)], dst_t)

    @pl.loop(0, N_PAD, step=L)
    def _(i):
        acc_t[pl.ds(i, L)] = jnp.zeros((L,), jnp.float32)

    ones_v = jnp.full((L,), 1.0, jnp.float32)

    @pl.loop(0, NGROUP)
    def _(g):
        idx = dst_t[pl.ds(g * L, L)]
        plsc.addupdate_scatter(acc_t, [idx], ones_v)

    pltpu.sync_copy(acc_t, shr.at[s])
    plsc.subcore_barrier()

    col0 = s * ROWS_PER_TILE
    pltpu.sync_copy(shr.at[:, pl.ds(col0, ROWS_PER_TILE)], red_in)

    @pl.loop(0, ROWS_PER_TILE, step=L)
    def _(w):
        tot = jnp.zeros((L,), jnp.float32)
        for r in range(NS):
            tot = tot + red_in[r, pl.ds(w, L)]
        red_out[pl.ds(w, L)] = tot

    pltpu.sync_copy(red_out, out_hbm.at[pl.ds(c * N_PAD + col0, ROWS_PER_TILE)])


GBLK = 128
NFULL = EDGES_PER_TILE // GBLK
TAIL = EDGES_PER_TILE - NFULL * GBLK
NSLOT = 2
NITER = NFULL // NSLOT - 1


@functools.partial(
    pl.kernel,
    out_type=jax.ShapeDtypeStruct((NC * N_PAD, D), jnp.float32),
    mesh=_sc_mesh,
    scratch_types=(
        [pltpu.VMEM((1, GBLK), jnp.int32)] * (2 * NSLOT)
        + [pltpu.VMEM((1, TAIL), jnp.int32)] * 2
        + [pltpu.VMEM((GBLK, D), jnp.float32)] * NSLOT
        + [pltpu.VMEM_SHARED((N_PAD, D), jnp.float32)]
        + [pltpu.SemaphoreType.DMA] * NSLOT
    ),
)
def _agg_sc(hs_hbm, src_hbm, dst_hbm, zeros_hbm, out_hbm,
            si0, di0, si1, di1, sit, dit,
            rows0, rows1, acc_sh, g0, g1):
    c = lax.axis_index("c")
    s = lax.axis_index("s")
    r0 = s * ROWS_PER_TILE
    pltpu.sync_copy(zeros_hbm.at[pl.ds(r0, ROWS_PER_TILE)],
                    acc_sh.at[pl.ds(r0, ROWS_PER_TILE)])
    plsc.subcore_barrier()
    base = c * EDGES_PER_CORE + s * EDGES_PER_TILE
    slots = ((si0, di0, rows0, g0), (si1, di1, rows1, g1))

    def fetch_idx(si, di, j):
        e0 = base + j * GBLK
        pltpu.sync_copy(src_hbm.at[pl.ds(e0, GBLK)], si.at[0])
        pltpu.sync_copy(dst_hbm.at[pl.ds(e0, GBLK)], di.at[0])

    def start_gather(si, rows, sem):
        pltpu.async_copy(hs_hbm.at[si.at[0]], rows, sem)

    def wait_gather(si, rows, sem):
        pltpu.make_async_copy(hs_hbm.at[si.at[0]], rows, sem).wait()

    for b, (si, di, rows, sem) in enumerate(slots):
        fetch_idx(si, di, b)
        start_gather(si, rows, sem)

    @pl.loop(0, NITER)
    def _(k):
        j = NSLOT * k
        for b, (si, di, rows, sem) in enumerate(slots):
            wait_gather(si, rows, sem)
            pltpu.sync_copy(rows, acc_sh.at[di.at[0]], add=True)
            fetch_idx(si, di, j + NSLOT + b)
            start_gather(si, rows, sem)

    for si, di, rows, sem in slots:
        wait_gather(si, rows, sem)
        pltpu.sync_copy(rows, acc_sh.at[di.at[0]], add=True)

    e0 = base + NFULL * GBLK
    pltpu.sync_copy(src_hbm.at[pl.ds(e0, TAIL)], sit.at[0])
    pltpu.sync_copy(dst_hbm.at[pl.ds(e0, TAIL)], dit.at[0])
    pltpu.async_copy(hs_hbm.at[sit.at[0]], rows0.at[pl.ds(0, TAIL)], g0).wait()
    pltpu.sync_copy(rows0.at[pl.ds(0, TAIL)], acc_sh.at[dit.at[0]], add=True)

    plsc.subcore_barrier()
    pltpu.sync_copy(acc_sh.at[pl.ds(r0, ROWS_PER_TILE)],
                    out_hbm.at[pl.ds(c * N_PAD + r0, ROWS_PER_TILE)])


R = 1280
GRID = -(-N_NODES // R)
OFF = N_PAD // R


def _hs1_body(x_ref, w_ref, p0_ref, p1_ref, hs_ref, dinv_ref):
    deg = p0_ref[...] + p1_ref[...] + 1.0
    dinv = lax.rsqrt(deg)
    dinv_ref[...] = dinv
    h = jnp.dot(x_ref[...], w_ref[...], preferred_element_type=jnp.float32)
    hs_ref[...] = h * dinv


_hs1_call = pl.pallas_call(
    _hs1_body,
    grid=(GRID,),
    in_specs=[
        pl.BlockSpec((R, D), lambda i: (i, 0)),
        pl.BlockSpec((D, D), lambda i: (0, 0)),
        pl.BlockSpec((R, 1), lambda i: (i, 0)),
        pl.BlockSpec((R, 1), lambda i: (OFF + i, 0)),
    ],
    out_specs=[
        pl.BlockSpec((R, D), lambda i: (i, 0)),
        pl.BlockSpec((R, 1), lambda i: (i, 0)),
    ],
    out_shape=[
        jax.ShapeDtypeStruct((N_NODES, D), jnp.float32),
        jax.ShapeDtypeStruct((N_NODES, 1), jnp.float32),
    ],
)


def _mid_body(a0_ref, a1_ref, hs_ref, dinv_ref, w_ref, b_ref, out_ref):
    dinv = dinv_ref[...]
    pre = dinv * (a0_ref[...] + a1_ref[...] + hs_ref[...]) + b_ref[...]
    f = jnp.maximum(pre, 0.0)
    out_ref[...] = jnp.dot(f, w_ref[...],
                           preferred_element_type=jnp.float32) * dinv


_mid_call = pl.pallas_call(
    _mid_body,
    grid=(GRID,),
    in_specs=[
        pl.BlockSpec((R, D), lambda i: (i, 0)),
        pl.BlockSpec((R, D), lambda i: (OFF + i, 0)),
        pl.BlockSpec((R, D), lambda i: (i, 0)),
        pl.BlockSpec((R, 1), lambda i: (i, 0)),
        pl.BlockSpec((D, D), lambda i: (0, 0)),
        pl.BlockSpec((1, D), lambda i: (0, 0)),
    ],
    out_specs=pl.BlockSpec((R, D), lambda i: (i, 0)),
    out_shape=jax.ShapeDtypeStruct((N_NODES, D), jnp.float32),
)


def _final_body(a0_ref, a1_ref, hs_ref, dinv_ref, b_ref, batch_ref,
                wfc_ref, bfc_ref, out_ref, pooled_ref, sums, cnts):
    i = pl.program_id(0)

    @pl.when(i == 0)
    def _init():
        sums[...] = jnp.zeros_like(sums)
        cnts[...] = jnp.zeros_like(cnts)

    dinv = dinv_ref[...]
    pre = dinv * (a0_ref[...] + a1_ref[...] + hs_ref[...]) + b_ref[...]
    h2 = jnp.maximum(pre, 0.0)
    rid = i * R + lax.broadcasted_iota(jnp.int32, (R, 1), 0)
    valid = rid < N_NODES
    h2 = jnp.where(valid, h2, 0.0)
    onehot = ((batch_ref[...] ==
               lax.broadcasted_iota(jnp.int32, (R, N_GRAPHS), 1)) & valid
              ).astype(jnp.float32)
    dn = (((0,), (0,)), ((), ()))
    sums[...] += lax.dot_general(onehot, h2, dn,
                                 preferred_element_type=jnp.float32)
    cnts[...] += lax.dot_general(onehot, jnp.ones((R, D), jnp.float32), dn,
                                 preferred_element_type=jnp.float32)

    @pl.when(i == pl.num_programs(0) - 1)
    def _fini():
        pooled = sums[...] / jnp.maximum(cnts[...], 1.0)
        pooled_ref[...] = pooled
        out_ref[...] = jnp.dot(pooled, wfc_ref[...],
                               preferred_element_type=jnp.float32) + bfc_ref[...]


_final_call = pl.pallas_call(
    _final_body,
    grid=(GRID,),
    in_specs=[
        pl.BlockSpec((R, D), lambda i: (i, 0)),
        pl.BlockSpec((R, D), lambda i: (OFF + i, 0)),
        pl.BlockSpec((R, D), lambda i: (i, 0)),
        pl.BlockSpec((R, 1), lambda i: (i, 0)),
        pl.BlockSpec((1, D), lambda i: (0, 0)),
        pl.BlockSpec((R, 1), lambda i: (i, 0)),
        pl.BlockSpec((D, CLS), lambda i: (0, 0)),
        pl.BlockSpec((1, CLS), lambda i: (0, 0)),
    ],
    out_specs=[
        pl.BlockSpec((N_GRAPHS, CLS), lambda i: (0, 0)),
        pl.BlockSpec((N_GRAPHS, D), lambda i: (0, 0)),
    ],
    out_shape=[
        jax.ShapeDtypeStruct((N_GRAPHS, CLS), jnp.float32),
        jax.ShapeDtypeStruct((N_GRAPHS, D), jnp.float32),
    ],
    scratch_shapes=[
        pltpu.VMEM((N_GRAPHS, D), jnp.float32),
        pltpu.VMEM((N_GRAPHS, D), jnp.float32),
    ],
)


def kernel(x, edge_index, batch, W1, b1, W2, b2, Wfc, bfc):
    src = edge_index[0].astype(jnp.int32)
    dst = edge_index[1].astype(jnp.int32)
    batch2 = batch.astype(jnp.int32).reshape(N_NODES, 1)
    zeros_nd = jnp.zeros((N_PAD, D), jnp.float32)

    degp = _degree_sc(dst).reshape(NC * N_PAD, 1)
    hs1, dinv = _hs1_call(x, W1, degp, degp)
    agg1 = _agg_sc(hs1, src, dst, zeros_nd)
    hs2 = _mid_call(agg1, agg1, hs1, dinv, W2, b1.reshape(1, D))
    agg2 = _agg_sc(hs2, src, dst, zeros_nd)
    output, pooled = _final_call(agg2, agg2, hs2, dinv,
                                 b2.reshape(1, D), batch2,
                                 Wfc, bfc.reshape(1, CLS))
    return (output, pooled)


# speedup vs baseline: 1.1647x; 1.1646x over previous
import dataclasses
import functools

import jax
import jax.numpy as jnp
from jax import lax
from jax.experimental import pallas as pl
from jax.experimental.pallas import tpu as pltpu
from jax.experimental.pallas import tpu_sc as plsc

N_NODES = 10000
N_EDGES = 320000
D = 128
CLS = 10
N_GRAPHS = 64

NC = 2
NS = 16
EDGES_PER_CORE = N_EDGES // NC
EDGES_PER_TILE = EDGES_PER_CORE // NS
BLK = 80
NBLK = EDGES_PER_TILE // BLK
N_PAD = 10240
ROWS_PER_TILE = N_PAD // NS

_sc_mesh = plsc.VectorSubcoreMesh(core_axis_name="c", subcore_axis_name="s")

_sc_cp = pltpu.CompilerParams()
if "needs_layout_passes" in pltpu.CompilerParams.__dataclass_fields__:
    _sc_cp = dataclasses.replace(_sc_cp, needs_layout_passes=False)


L = 16
NGROUP = EDGES_PER_TILE // L


@functools.partial(
    pl.kernel,
    out_type=jax.ShapeDtypeStruct((NC * N_PAD,), jnp.float32),
    mesh=_sc_mesh,
    compiler_params=_sc_cp,
    scratch_types=[
        pltpu.VMEM((EDGES_PER_TILE,), jnp.int32),
        pltpu.VMEM((N_PAD,), jnp.float32),
        pltpu.VMEM((L, ROWS_PER_TILE), jnp.float32),
        pltpu.VMEM((ROWS_PER_TILE,), jnp.float32),
        pltpu.VMEM_SHARED((NS, N_PAD), jnp.float32),
    ],
)
def _degree_sc(dst_hbm, out_hbm, dst_t, acc_t, red_in, red_out, shr):
    c = lax.axis_index("c")
    s = lax.axis_index("s")
    base = c * EDGES_PER_CORE + s * EDGES_PER_TILE
    pltpu.sync_copy(dst_hbm.at[pl.ds(base, EDGES_PER_TILE)], dst_t)

    @pl.loop(0, N_PAD, step=L)
    def _(i):
        acc_t[pl.ds(i, L)] = jnp.zeros((L,), jnp.float32)

    ones_v = jnp.full((L,), 1.0, jnp.float32)

    @pl.loop(0, NGROUP)
    def _(g):
        idx = dst_t[pl.ds(g * L, L)]
        plsc.addupdate_scatter(acc_t, [idx], ones_v)

    pltpu.sync_copy(acc_t, shr.at[s])
    plsc.subcore_barrier()

    col0 = s * ROWS_PER_TILE
    pltpu.sync_copy(shr.at[:, pl.ds(col0, ROWS_PER_TILE)], red_in)

    @pl.loop(0, ROWS_PER_TILE, step=L)
    def _(w):
        tot = jnp.zeros((L,), jnp.float32)
        for r in range(NS):
            tot = tot + red_in[r, pl.ds(w, L)]
        red_out[pl.ds(w, L)] = tot

    pltpu.sync_copy(red_out, out_hbm.at[pl.ds(c * N_PAD + col0, ROWS_PER_TILE)])


GBLK = 120
NFULL = EDGES_PER_TILE // GBLK
TAIL = EDGES_PER_TILE - NFULL * GBLK
NSLOT = 3
NITER = (NFULL - NSLOT - 2) // NSLOT


@functools.partial(
    pl.kernel,
    out_type=jax.ShapeDtypeStruct((NC * N_PAD, D), jnp.float32),
    mesh=_sc_mesh,
    scratch_types=(
        [pltpu.VMEM((1, GBLK), jnp.int32)] * (2 * NSLOT)
        + [pltpu.VMEM((1, TAIL), jnp.int32)] * 2
        + [pltpu.VMEM((GBLK, D), jnp.float32)] * NSLOT
        + [pltpu.VMEM_SHARED((N_PAD, D), jnp.float32)]
        + [pltpu.SemaphoreType.DMA] * (2 * NSLOT)
    ),
)
def _agg_sc(hs_hbm, src_hbm, dst_hbm, zeros_hbm, out_hbm,
            si0, di0, si1, di1, si2, di2, sit, dit,
            rows0, rows1, rows2, acc_sh, g0, g1, g2, s0, s1, s2):
    c = lax.axis_index("c")
    s = lax.axis_index("s")
    r0 = s * ROWS_PER_TILE
    pltpu.sync_copy(zeros_hbm.at[pl.ds(r0, ROWS_PER_TILE)],
                    acc_sh.at[pl.ds(r0, ROWS_PER_TILE)])
    plsc.subcore_barrier()
    base = c * EDGES_PER_CORE + s * EDGES_PER_TILE
    slots = ((si0, di0, rows0, g0, s0),
             (si1, di1, rows1, g1, s1),
             (si2, di2, rows2, g2, s2))

    def fetch_idx(si, di, j):
        e0 = base + j * GBLK
        pltpu.sync_copy(src_hbm.at[pl.ds(e0, GBLK)], si.at[0])
        pltpu.sync_copy(dst_hbm.at[pl.ds(e0, GBLK)], di.at[0])

    def start_gather(si, rows, sem):
        pltpu.async_copy(hs_hbm.at[si.at[0]], rows, sem)

    def wait_gather(si, rows, sem):
        pltpu.make_async_copy(hs_hbm.at[si.at[0]], rows, sem).wait()

    def start_scatter(rows, di, sem):
        pltpu.async_copy(rows, acc_sh.at[di.at[0]], sem, add=True)

    def drain_scatter(rows, sem):
        pltpu.make_async_copy(zeros_hbm.at[pl.ds(0, GBLK)], rows, sem).wait()

    for b, (si, di, rows, g, _) in enumerate(slots):
        fetch_idx(si, di, b)
        start_gather(si, rows, g)
    wait_gather(si0, rows0, g0)
    start_scatter(rows0, di0, s0)
    wait_gather(si1, rows1, g1)
    start_scatter(rows1, di1, s1)

    def group(j, b):
        si, di, rows, g, sc = slots[b]
        sip, dip, rowsp, gp, scp = slots[(b - 1) % NSLOT]
        drain_scatter(rows, sc)
        fetch_idx(si, di, j)
        start_gather(si, rows, g)
        wait_gather(sip, rowsp, gp)
        start_scatter(rowsp, dip, scp)

    @pl.loop(0, NITER)
    def _(k):
        j = NSLOT * k + NSLOT
        group(j, 0)
        group(j + 1, 1)
        group(j + 2, 2)

    group(NFULL - 2, 0)
    group(NFULL - 1, 1)
    wait_gather(si1, rows1, g1)
    start_scatter(rows1, di1, s1)
    drain_scatter(rows2, s2)
    drain_scatter(rows0, s0)
    drain_scatter(rows1, s1)

    e0 = base + NFULL * GBLK
    pltpu.sync_copy(src_hbm.at[pl.ds(e0, TAIL)], sit.at[0])
    pltpu.sync_copy(dst_hbm.at[pl.ds(e0, TAIL)], dit.at[0])
    pltpu.async_copy(hs_hbm.at[sit.at[0]], rows0.at[pl.ds(0, TAIL)], g0).wait()
    pltpu.sync_copy(rows0.at[pl.ds(0, TAIL)], acc_sh.at[dit.at[0]], add=True)

    plsc.subcore_barrier()
    pltpu.sync_copy(acc_sh.at[pl.ds(r0, ROWS_PER_TILE)],
                    out_hbm.at[pl.ds(c * N_PAD + r0, ROWS_PER_TILE)])


R = 1280
GRID = -(-N_NODES // R)
OFF = N_PAD // R


def _hs1_body(x_ref, w_ref, p0_ref, p1_ref, hs_ref, dinv_ref):
    deg = p0_ref[...] + p1_ref[...] + 1.0
    dinv = lax.rsqrt(deg)
    dinv_ref[...] = dinv
    h = jnp.dot(x_ref[...], w_ref[...], preferred_element_type=jnp.float32)
    hs_ref[...] = h * dinv


_hs1_call = pl.pallas_call(
    _hs1_body,
    grid=(GRID,),
    in_specs=[
        pl.BlockSpec((R, D), lambda i: (i, 0)),
        pl.BlockSpec((D, D), lambda i: (0, 0)),
        pl.BlockSpec((R, 1), lambda i: (i, 0)),
        pl.BlockSpec((R, 1), lambda i: (OFF + i, 0)),
    ],
    out_specs=[
        pl.BlockSpec((R, D), lambda i: (i, 0)),
        pl.BlockSpec((R, 1), lambda i: (i, 0)),
    ],
    out_shape=[
        jax.ShapeDtypeStruct((N_NODES, D), jnp.float32),
        jax.ShapeDtypeStruct((N_NODES, 1), jnp.float32),
    ],
)


def _mid_body(a0_ref, a1_ref, hs_ref, dinv_ref, w_ref, b_ref, out_ref):
    dinv = dinv_ref[...]
    pre = dinv * (a0_ref[...] + a1_ref[...] + hs_ref[...]) + b_ref[...]
    f = jnp.maximum(pre, 0.0)
    out_ref[...] = jnp.dot(f, w_ref[...],
                           preferred_element_type=jnp.float32) * dinv


_mid_call = pl.pallas_call(
    _mid_body,
    grid=(GRID,),
    in_specs=[
        pl.BlockSpec((R, D), lambda i: (i, 0)),
        pl.BlockSpec((R, D), lambda i: (OFF + i, 0)),
        pl.BlockSpec((R, D), lambda i: (i, 0)),
        pl.BlockSpec((R, 1), lambda i: (i, 0)),
        pl.BlockSpec((D, D), lambda i: (0, 0)),
        pl.BlockSpec((1, D), lambda i: (0, 0)),
    ],
    out_specs=pl.BlockSpec((R, D), lambda i: (i, 0)),
    out_shape=jax.ShapeDtypeStruct((N_NODES, D), jnp.float32),
)


def _final_body(a0_ref, a1_ref, hs_ref, dinv_ref, b_ref, batch_ref,
                wfc_ref, bfc_ref, out_ref, pooled_ref, sums, cnts):
    i = pl.program_id(0)

    @pl.when(i == 0)
    def _init():
        sums[...] = jnp.zeros_like(sums)
        cnts[...] = jnp.zeros_like(cnts)

    dinv = dinv_ref[...]
    pre = dinv * (a0_ref[...] + a1_ref[...] + hs_ref[...]) + b_ref[...]
    h2 = jnp.maximum(pre, 0.0)
    rid = i * R + lax.broadcasted_iota(jnp.int32, (R, 1), 0)
    valid = rid < N_NODES
    h2 = jnp.where(valid, h2, 0.0)
    onehot = ((batch_ref[...] ==
               lax.broadcasted_iota(jnp.int32, (R, N_GRAPHS), 1)) & valid
              ).astype(jnp.float32)
    dn = (((0,), (0,)), ((), ()))
    sums[...] += lax.dot_general(onehot, h2, dn,
                                 preferred_element_type=jnp.float32)
    cnts[...] += lax.dot_general(onehot, jnp.ones((R, D), jnp.float32), dn,
                                 preferred_element_type=jnp.float32)

    @pl.when(i == pl.num_programs(0) - 1)
    def _fini():
        pooled = sums[...] / jnp.maximum(cnts[...], 1.0)
        pooled_ref[...] = pooled
        out_ref[...] = jnp.dot(pooled, wfc_ref[...],
                               preferred_element_type=jnp.float32) + bfc_ref[...]


_final_call = pl.pallas_call(
    _final_body,
    grid=(GRID,),
    in_specs=[
        pl.BlockSpec((R, D), lambda i: (i, 0)),
        pl.BlockSpec((R, D), lambda i: (OFF + i, 0)),
        pl.BlockSpec((R, D), lambda i: (i, 0)),
        pl.BlockSpec((R, 1), lambda i: (i, 0)),
        pl.BlockSpec((1, D), lambda i: (0, 0)),
        pl.BlockSpec((R, 1), lambda i: (i, 0)),
        pl.BlockSpec((D, CLS), lambda i: (0, 0)),
        pl.BlockSpec((1, CLS), lambda i: (0, 0)),
    ],
    out_specs=[
        pl.BlockSpec((N_GRAPHS, CLS), lambda i: (0, 0)),
        pl.BlockSpec((N_GRAPHS, D), lambda i: (0, 0)),
    ],
    out_shape=[
        jax.ShapeDtypeStruct((N_GRAPHS, CLS), jnp.float32),
        jax.ShapeDtypeStruct((N_GRAPHS, D), jnp.float32),
    ],
    scratch_shapes=[
        pltpu.VMEM((N_GRAPHS, D), jnp.float32),
        pltpu.VMEM((N_GRAPHS, D), jnp.float32),
    ],
)


def kernel(x, edge_index, batch, W1, b1, W2, b2, Wfc, bfc):
    src = edge_index[0].astype(jnp.int32)
    dst = edge_index[1].astype(jnp.int32)
    batch2 = batch.astype(jnp.int32).reshape(N_NODES, 1)
    zeros_nd = jnp.zeros((N_PAD, D), jnp.float32)

    degp = _degree_sc(dst).reshape(NC * N_PAD, 1)
    hs1, dinv = _hs1_call(x, W1, degp, degp)
    agg1 = _agg_sc(hs1, src, dst, zeros_nd)
    hs2 = _mid_call(agg1, agg1, hs1, dinv, W2, b1.reshape(1, D))
    agg2 = _agg_sc(hs2, src, dst, zeros_nd)
    output, pooled = _final_call(agg2, agg2, hs2, dinv,
                                 b2.reshape(1, D), batch2,
                                 Wfc, bfc.reshape(1, CLS))
    return (output, pooled)


# speedup vs baseline: 1.2734x; 1.0934x over previous
import dataclasses
import functools

import jax
import jax.numpy as jnp
from jax import lax
from jax.experimental import pallas as pl
from jax.experimental.pallas import tpu as pltpu
from jax.experimental.pallas import tpu_sc as plsc

N_NODES = 10000
N_EDGES = 320000
D = 128
CLS = 10
N_GRAPHS = 64

NC = 2
NS = 16
EDGES_PER_CORE = N_EDGES // NC
EDGES_PER_TILE = EDGES_PER_CORE // NS
BLK = 80
NBLK = EDGES_PER_TILE // BLK
N_PAD = 10240
ROWS_PER_TILE = N_PAD // NS

_sc_mesh = plsc.VectorSubcoreMesh(core_axis_name="c", subcore_axis_name="s")

_sc_cp = pltpu.CompilerParams()
if "needs_layout_passes" in pltpu.CompilerParams.__dataclass_fields__:
    _sc_cp = dataclasses.replace(_sc_cp, needs_layout_passes=False)


L = 16
NGROUP = EDGES_PER_TILE // L


@functools.partial(
    pl.kernel,
    out_type=jax.ShapeDtypeStruct((NC * N_PAD,), jnp.float32),
    mesh=_sc_mesh,
    compiler_params=_sc_cp,
    scratch_types=[
        pltpu.VMEM((EDGES_PER_TILE,), jnp.int32),
        pltpu.VMEM((N_PAD,), jnp.float32),
        pltpu.VMEM((L, ROWS_PER_TILE), jnp.float32),
        pltpu.VMEM((ROWS_PER_TILE,), jnp.float32),
        pltpu.VMEM_SHARED((NS, N_PAD), jnp.float32),
    ],
)
def _degree_sc(dst_hbm, out_hbm, dst_t, acc_t, red_in, red_out, shr):
    c = lax.axis_index("c")
    s = lax.axis_index("s")
    base = c * EDGES_PER_CORE + s * EDGES_PER_TILE
    pltpu.sync_copy(dst_hbm.at[pl.ds(base, EDGES_PER_TILE)], dst_t)

    @pl.loop(0, N_PAD, step=L)
    def _(i):
        acc_t[pl.ds(i, L)] = jnp.zeros((L,), jnp.float32)

    ones_v = jnp.full((L,), 1.0, jnp.float32)

    @pl.loop(0, NGROUP)
    def _(g):
        idx = dst_t[pl.ds(g * L, L)]
        plsc.addupdate_scatter(acc_t, [idx], ones_v)

    pltpu.sync_copy(acc_t, shr.at[s])
    plsc.subcore_barrier()

    col0 = s * ROWS_PER_TILE
    pltpu.sync_copy(shr.at[:, pl.ds(col0, ROWS_PER_TILE)], red_in)

    @pl.loop(0, ROWS_PER_TILE, step=L)
    def _(w):
        tot = jnp.zeros((L,), jnp.float32)
        for r in range(NS):
            tot = tot + red_in[r, pl.ds(w, L)]
        red_out[pl.ds(w, L)] = tot

    pltpu.sync_copy(red_out, out_hbm.at[pl.ds(c * N_PAD + col0, ROWS_PER_TILE)])


GBLK = 120
NFULL = EDGES_PER_TILE // GBLK
TAIL = EDGES_PER_TILE - NFULL * GBLK
NSLOT = 3
NITER = (NFULL - NSLOT - 2) // NSLOT


@functools.partial(
    pl.kernel,
    out_type=jax.ShapeDtypeStruct((NC * N_PAD, D), jnp.float32),
    mesh=_sc_mesh,
    scratch_types=(
        [pltpu.VMEM((2, GBLK), jnp.int32)] * NSLOT
        + [pltpu.VMEM((2, TAIL), jnp.int32)]
        + [pltpu.VMEM((GBLK, D), jnp.float32)] * NSLOT
        + [pltpu.VMEM_SHARED((N_PAD, D), jnp.float32)]
        + [pltpu.SemaphoreType.DMA] * (2 * NSLOT)
    ),
)
def _agg_sc(hs_hbm, pf_hbm, pt_hbm, zeros_hbm, out_hbm,
            ix0, ix1, ix2, ixt,
            rows0, rows1, rows2, acc_sh, g0, g1, g2, s0, s1, s2):
    c = lax.axis_index("c")
    s = lax.axis_index("s")
    r0 = s * ROWS_PER_TILE
    pltpu.sync_copy(zeros_hbm.at[pl.ds(r0, ROWS_PER_TILE)],
                    acc_sh.at[pl.ds(r0, ROWS_PER_TILE)])
    plsc.subcore_barrier()
    w = c * NS + s
    slots = ((ix0, rows0, g0, s0), (ix1, rows1, g1, s1), (ix2, rows2, g2, s2))

    def fetch_idx(ix, j):
        pltpu.sync_copy(pf_hbm.at[pl.ds(2 * (w * NFULL + j), 2)], ix)

    def start_gather(ix, rows, sem):
        pltpu.async_copy(hs_hbm.at[ix.at[0]], rows, sem)

    def wait_gather(ix, rows, sem):
        pltpu.make_async_copy(hs_hbm.at[ix.at[0]], rows, sem).wait()

    def start_scatter(rows, ix, sem):
        pltpu.async_copy(rows, acc_sh.at[ix.at[1]], sem, add=True)

    def drain_scatter(rows, sem):
        pltpu.make_async_copy(zeros_hbm.at[pl.ds(0, GBLK)], rows, sem).wait()

    for b, (ix, rows, g, _) in enumerate(slots):
        fetch_idx(ix, b)
        start_gather(ix, rows, g)
    wait_gather(ix0, rows0, g0)
    start_scatter(rows0, ix0, s0)
    wait_gather(ix1, rows1, g1)
    start_scatter(rows1, ix1, s1)

    def group(j, b):
        ix, rows, g, sc = slots[b]
        ixp, rowsp, gp, scp = slots[(b - 1) % NSLOT]
        drain_scatter(rows, sc)
        fetch_idx(ix, j)
        start_gather(ix, rows, g)
        wait_gather(ixp, rowsp, gp)
        start_scatter(rowsp, ixp, scp)

    @pl.loop(0, NITER)
    def _(k):
        j = NSLOT * k + NSLOT
        group(j, 0)
        group(j + 1, 1)
        group(j + 2, 2)

    group(NFULL - 2, 0)
    group(NFULL - 1, 1)
    wait_gather(ix1, rows1, g1)
    start_scatter(rows1, ix1, s1)
    drain_scatter(rows2, s2)
    drain_scatter(rows0, s0)
    drain_scatter(rows1, s1)

    pltpu.sync_copy(pt_hbm.at[pl.ds(2 * w, 2)], ixt)
    pltpu.async_copy(hs_hbm.at[ixt.at[0]], rows0.at[pl.ds(0, TAIL)], g0).wait()
    pltpu.sync_copy(rows0.at[pl.ds(0, TAIL)], acc_sh.at[ixt.at[1]], add=True)

    plsc.subcore_barrier()
    pltpu.sync_copy(acc_sh.at[pl.ds(r0, ROWS_PER_TILE)],
                    out_hbm.at[pl.ds(c * N_PAD + r0, ROWS_PER_TILE)])


R = 1280
GRID = -(-N_NODES // R)
OFF = N_PAD // R


def _hs1_body(x_ref, w_ref, p0_ref, p1_ref, hs_ref, dinv_ref):
    deg = p0_ref[...] + p1_ref[...] + 1.0
    dinv = lax.rsqrt(deg)
    dinv_ref[...] = dinv
    h = jnp.dot(x_ref[...], w_ref[...], preferred_element_type=jnp.float32)
    hs_ref[...] = h * dinv


_hs1_call = pl.pallas_call(
    _hs1_body,
    grid=(GRID,),
    in_specs=[
        pl.BlockSpec((R, D), lambda i: (i, 0)),
        pl.BlockSpec((D, D), lambda i: (0, 0)),
        pl.BlockSpec((R, 1), lambda i: (i, 0)),
        pl.BlockSpec((R, 1), lambda i: (OFF + i, 0)),
    ],
    out_specs=[
        pl.BlockSpec((R, D), lambda i: (i, 0)),
        pl.BlockSpec((R, 1), lambda i: (i, 0)),
    ],
    out_shape=[
        jax.ShapeDtypeStruct((N_NODES, D), jnp.float32),
        jax.ShapeDtypeStruct((N_NODES, 1), jnp.float32),
    ],
)


def _mid_body(a0_ref, a1_ref, hs_ref, dinv_ref, w_ref, b_ref, out_ref):
    dinv = dinv_ref[...]
    pre = dinv * (a0_ref[...] + a1_ref[...] + hs_ref[...]) + b_ref[...]
    f = jnp.maximum(pre, 0.0)
    out_ref[...] = jnp.dot(f, w_ref[...],
                           preferred_element_type=jnp.float32) * dinv


_mid_call = pl.pallas_call(
    _mid_body,
    grid=(GRID,),
    in_specs=[
        pl.BlockSpec((R, D), lambda i: (i, 0)),
        pl.BlockSpec((R, D), lambda i: (OFF + i, 0)),
        pl.BlockSpec((R, D), lambda i: (i, 0)),
        pl.BlockSpec((R, 1), lambda i: (i, 0)),
        pl.BlockSpec((D, D), lambda i: (0, 0)),
        pl.BlockSpec((1, D), lambda i: (0, 0)),
    ],
    out_specs=pl.BlockSpec((R, D), lambda i: (i, 0)),
    out_shape=jax.ShapeDtypeStruct((N_NODES, D), jnp.float32),
)


def _final_body(a0_ref, a1_ref, hs_ref, dinv_ref, b_ref, batch_ref,
                wfc_ref, bfc_ref, out_ref, pooled_ref, sums, cnts):
    i = pl.program_id(0)

    @pl.when(i == 0)
    def _init():
        sums[...] = jnp.zeros_like(sums)
        cnts[...] = jnp.zeros_like(cnts)

    dinv = dinv_ref[...]
    pre = dinv * (a0_ref[...] + a1_ref[...] + hs_ref[...]) + b_ref[...]
    h2 = jnp.maximum(pre, 0.0)
    rid = i * R + lax.broadcasted_iota(jnp.int32, (R, 1), 0)
    valid = rid < N_NODES
    h2 = jnp.where(valid, h2, 0.0)
    onehot = ((batch_ref[...] ==
               lax.broadcasted_iota(jnp.int32, (R, N_GRAPHS), 1)) & valid
              ).astype(jnp.float32)
    dn = (((0,), (0,)), ((), ()))
    sums[...] += lax.dot_general(onehot, h2, dn,
                                 preferred_element_type=jnp.float32)
    cnts[...] += lax.dot_general(onehot, jnp.ones((R, D), jnp.float32), dn,
                                 preferred_element_type=jnp.float32)

    @pl.when(i == pl.num_programs(0) - 1)
    def _fini():
        pooled = sums[...] / jnp.maximum(cnts[...], 1.0)
        pooled_ref[...] = pooled
        out_ref[...] = jnp.dot(pooled, wfc_ref[...],
                               preferred_element_type=jnp.float32) + bfc_ref[...]


_final_call = pl.pallas_call(
    _final_body,
    grid=(GRID,),
    in_specs=[
        pl.BlockSpec((R, D), lambda i: (i, 0)),
        pl.BlockSpec((R, D), lambda i: (OFF + i, 0)),
        pl.BlockSpec((R, D), lambda i: (i, 0)),
        pl.BlockSpec((R, 1), lambda i: (i, 0)),
        pl.BlockSpec((1, D), lambda i: (0, 0)),
        pl.BlockSpec((R, 1), lambda i: (i, 0)),
        pl.BlockSpec((D, CLS), lambda i: (0, 0)),
        pl.BlockSpec((1, CLS), lambda i: (0, 0)),
    ],
    out_specs=[
        pl.BlockSpec((N_GRAPHS, CLS), lambda i: (0, 0)),
        pl.BlockSpec((N_GRAPHS, D), lambda i: (0, 0)),
    ],
    out_shape=[
        jax.ShapeDtypeStruct((N_GRAPHS, CLS), jnp.float32),
        jax.ShapeDtypeStruct((N_GRAPHS, D), jnp.float32),
    ],
    scratch_shapes=[
        pltpu.VMEM((N_GRAPHS, D), jnp.float32),
        pltpu.VMEM((N_GRAPHS, D), jnp.float32),
    ],
)


NW = NC * NS


def _pack_edge_blocks(ei32):
    per_tile = ei32.reshape(2, NW, EDGES_PER_TILE)
    full = per_tile[:, :, :NFULL * GBLK].reshape(2, NW, NFULL, GBLK)
    pf = jnp.transpose(full, (1, 2, 0, 3)).reshape(NW * NFULL * 2, GBLK)
    tail = per_tile[:, :, NFULL * GBLK:]
    pt = jnp.transpose(tail, (1, 0, 2)).reshape(NW * 2, TAIL)
    return pf, pt


def kernel(x, edge_index, batch, W1, b1, W2, b2, Wfc, bfc):
    ei32 = edge_index.astype(jnp.int32)
    dst = ei32[1]
    pf, pt = _pack_edge_blocks(ei32)
    batch2 = batch.astype(jnp.int32).reshape(N_NODES, 1)
    zeros_nd = jnp.zeros((N_PAD, D), jnp.float32)

    degp = _degree_sc(dst).reshape(NC * N_PAD, 1)
    hs1, dinv = _hs1_call(x, W1, degp, degp)
    agg1 = _agg_sc(hs1, pf, pt, zeros_nd)
    hs2 = _mid_call(agg1, agg1, hs1, dinv, W2, b1.reshape(1, D))
    agg2 = _agg_sc(hs2, pf, pt, zeros_nd)
    output, pooled = _final_call(agg2, agg2, hs2, dinv,
                                 b2.reshape(1, D), batch2,
                                 Wfc, bfc.reshape(1, CLS))
    return (output, pooled)
